# baseline (speedup 1.0000x reference)
import functools

import jax
import jax.numpy as jnp
from jax import lax
from jax.experimental import pallas as pl
from jax.experimental.pallas import tpu as pltpu
from jax.experimental.pallas import tpu_sc as plsc

_CHUNK = 128
_SHALF = 32


def _sc_dims():
    try:
        info = plsc.get_sparse_core_info()
        return info.num_cores, info.num_subcores
    except Exception:
        return 2, 16


def _fill_1d(ref, n, value):
    v = jnp.full((16,), value, jnp.float32)

    def body(j, c):
        ref[pl.ds(j * 16, 16)] = v
        return c

    lax.fori_loop(0, n // 16, body, 0)


def _zero_2d(ref, rows, cols):
    z = jnp.zeros((16,), jnp.float32)
    per_row = cols // 16

    def body(j, c):
        ref[j // per_row, pl.ds((j % per_row) * 16, 16)] = z
        return c

    lax.fori_loop(0, rows * per_row, body, 0)


def _make_fused_kernel(E, N, n_pad, D):
    NC, NS = _sc_dims()
    NW = NC * NS
    dchunks = ((E // NS + 16 * _CHUNK - 1) // (16 * _CHUNK)) * 16
    dstage = dchunks // 2
    n_chunks = ((E // NW + 8 * _CHUNK - 1) // (8 * _CHUNK)) * 8
    stripe = n_pad // NS
    xrows = 128
    xblocks = stripe // xrows
    nx = N + 8
    mesh = plsc.VectorSubcoreMesh(core_axis_name="c", subcore_axis_name="s")

    @functools.partial(
        pl.kernel,
        out_type=[
            jax.ShapeDtypeStruct((NC, n_pad, D), jnp.float32),
            jax.ShapeDtypeStruct((NC * nx, D), jnp.float32),
            jax.ShapeDtypeStruct((n_pad,), jnp.float32),
        ],
        mesh=mesh,
        scratch_types=[
            pltpu.VMEM((_SHALF, _CHUNK), jnp.int32),
            pltpu.VMEM((max(n_chunks, dstage), _CHUNK), jnp.int32),
            pltpu.VMEM((2, _CHUNK, D), jnp.float32),
            pltpu.VMEM((stripe + 16,), jnp.float32),
            pltpu.VMEM((_CHUNK,), jnp.float32),
            pltpu.VMEM_SHARED((n_pad, D), jnp.float32),
            pltpu.VMEM_SHARED((n_pad,), jnp.float32),
            pltpu.SemaphoreType.DMA,
            pltpu.SemaphoreType.DMA,
            pltpu.SemaphoreType.DMA,
            pltpu.SemaphoreType.DMA,
        ],
    )
    def fused_kernel(x_hbm, dstp_hbm, src3_hbm, dst3_hbm,
                     acc_out, xs_out, dinv_out,
                     sidx_v, didx_v, rows_v, dbuf_v, ones_v,
                     acc_sh, deg_sh, gsem0, gsem1, ssem0, ssem1):
        cid = lax.axis_index("c")
        sid = lax.axis_index("s")
        wid = cid * NS + sid
        row0 = sid * stripe
        xoff = cid * nx

        _fill_1d(ones_v, _CHUNK, 1.0)
        _fill_1d(dbuf_v, stripe + 16, 0.0)
        pltpu.sync_copy(dbuf_v.at[pl.ds(0, stripe)],
                        deg_sh.at[pl.ds(sid * stripe, stripe)])
        zrows = 40
        _zero_2d(rows_v.at[0], zrows, D)

        def zbody(r, c):
            pltpu.sync_copy(
                rows_v.at[0, pl.ds(0, zrows)],
                acc_sh.at[pl.ds(row0 + r * zrows, zrows)],
            )
            return c

        lax.fori_loop(0, stripe // zrows, zbody, 0)
        plsc.subcore_barrier()

        def dfire(i, c):
            pltpu.async_copy(ones_v, deg_sh.at[didx_v.at[i]], ssem0, add=True)
            return c

        def ddrain(i, c):
            pltpu.make_async_copy(ones_v, deg_sh.at[didx_v.at[0]], ssem0).wait()
            return c

        pltpu.sync_copy(dstp_hbm.at[sid, pl.ds(0, dstage)],
                        didx_v.at[pl.ds(0, dstage)])
        lax.fori_loop(0, dstage, dfire, 0)
        lax.fori_loop(0, dstage, ddrain, 0)
        pltpu.sync_copy(dstp_hbm.at[sid, pl.ds(dstage, dchunks - dstage)],
                        didx_v.at[pl.ds(0, dchunks - dstage)])
        lax.fori_loop(0, dchunks - dstage, dfire, 0)
        lax.fori_loop(0, dchunks - dstage, ddrain, 0)
        plsc.subcore_barrier()

        pltpu.sync_copy(deg_sh.at[pl.ds(sid * stripe, stripe)],
                        dbuf_v.at[pl.ds(0, stripe)])

        def newton(j, c):
            d = dbuf_v[pl.ds(j * 16, 16)] + 1.0
            bits = lax.bitcast_convert_type(d, jnp.int32)
            y = lax.bitcast_convert_type(
                jnp.int32(0x5F3759DF) - (bits >> 1), jnp.float32
            )
            y = y * (1.5 - 0.5 * d * y * y)
            y = y * (1.5 - 0.5 * d * y * y)
            y = y * (1.5 - 0.5 * d * y * y)
            dbuf_v[pl.ds(j * 16, 16)] = y
            return c

        lax.fori_loop(0, stripe // 16, newton, 0)

        @pl.when(cid == 0)
        def _():
            pltpu.sync_copy(
                dbuf_v.at[pl.ds(0, stripe)],
                dinv_out.at[pl.ds(sid * stripe, stripe)],
            )

        def xblock(b, c):
            rowstart = jnp.minimum(sid * stripe + b * xrows, N - xrows)
            pltpu.sync_copy(x_hbm.at[pl.ds(rowstart, xrows)], rows_v.at[0])
            dbase = rowstart - sid * stripe

            def srow(r, c2):
                y = dbuf_v[pl.ds(dbase + r, 16)][0]
                for j in range(D // 16):
                    rows_v[0, r, pl.ds(j * 16, 16)] = (
                        rows_v[0, r, pl.ds(j * 16, 16)] * y
                    )
                return c2

            lax.fori_loop(0, xrows, srow, 0)
            pltpu.sync_copy(
                rows_v.at[0], xs_out.at[pl.ds(xoff + rowstart, xrows)]
            )
            return c

        lax.fori_loop(0, xblocks, xblock, 0)

        @pl.when(sid == NS - 1)
        def _():
            _zero_2d(rows_v.at[0], 8, D)
            pltpu.sync_copy(
                rows_v.at[0, pl.ds(0, 8)], xs_out.at[pl.ds(xoff + N, 8)]
            )

        plsc.subcore_barrier()

        def stage_src(lo, cnt):
            pltpu.sync_copy(
                src3_hbm.at[wid, pl.ds(lo, cnt)], sidx_v.at[pl.ds(0, cnt)]
            )

            def adj(jj, c):
                sl = pl.ds((jj % (_CHUNK // 16)) * 16, 16)
                sidx_v[jj // (_CHUNK // 16), sl] = (
                    sidx_v[jj // (_CHUNK // 16), sl] + xoff
                )
                return c

            lax.fori_loop(0, cnt * (_CHUNK // 16), adj, 0)

        stage_src(0, min(_SHALF, n_chunks))
        pltpu.sync_copy(dst3_hbm.at[wid], didx_v.at[pl.ds(0, n_chunks)])

        def gfire(i, buf, gsem):
            pltpu.async_copy(
                xs_out.at[sidx_v.at[i % _SHALF]], rows_v.at[buf], gsem
            )

        def gwait(buf, gsem):
            pltpu.make_async_copy(
                xs_out.at[sidx_v.at[0]], rows_v.at[buf], gsem
            ).wait()

        def sfire(i, buf, ssem):
            pltpu.async_copy(
                rows_v.at[buf], acc_sh.at[didx_v.at[i]], ssem, add=True
            )

        def sdrain(buf, ssem):
            pltpu.make_async_copy(
                rows_v.at[buf], acc_sh.at[didx_v.at[0]], ssem
            ).wait()

        def halfstep(i, cur, cgsem, cssem, nxt, ngsem, nssem):
            @pl.when(i >= 1)
            def _():
                sdrain(nxt, nssem)

            @pl.when((i + 1 < n_chunks) & ((i + 1) % _SHALF != 0))
            def _():
                gfire(i + 1, nxt, ngsem)

            gwait(cur, cgsem)
            sfire(i, cur, cssem)

            @pl.when(((i + 1) % _SHALF == 0) & (i + 1 < n_chunks))
            def _():
                for lo in range(_SHALF, n_chunks, _SHALF):
                    @pl.when(i + 1 == lo)
                    def _():
                        stage_src(lo, min(_SHALF, n_chunks - lo))

                gfire(i + 1, nxt, ngsem)

        gfire(0, 0, gsem0)

        def body(k, c):
            halfstep(2 * k, 0, gsem0, ssem0, 1, gsem1, ssem1)

            @pl.when(2 * k + 1 < n_chunks)
            def _():
                halfstep(2 * k + 1, 1, gsem1, ssem1, 0, gsem0, ssem0)

            return c

        lax.fori_loop(0, (n_chunks + 1) // 2, body, 0)
        sdrain((n_chunks - 1) % 2, ssem1 if (n_chunks - 1) % 2 == 1 else ssem0)
        plsc.subcore_barrier()
        pltpu.sync_copy(
            acc_sh.at[pl.ds(row0, stripe)],
            acc_out.at[cid, pl.ds(row0, stripe)],
        )

    return fused_kernel


def _final_body(acc_ref, xs_ref, dinv_ref, w_ref, b_ref, out_ref):
    h = (acc_ref[0] + acc_ref[1] + xs_ref[...]) * dinv_ref[...]
    out_ref[...] = (
        jnp.dot(h, w_ref[...], preferred_element_type=jnp.float32) + b_ref[...]
    )


def _make_final_kernel(N, NX2, D, block_rows):
    return pl.pallas_call(
        _final_body,
        grid=(N // block_rows,),
        in_specs=[
            pl.BlockSpec((2, block_rows, D), lambda i: (0, i, 0)),
            pl.BlockSpec((block_rows, D), lambda i: (i, 0)),
            pl.BlockSpec((block_rows, 1), lambda i: (i, 0)),
            pl.BlockSpec((D, D), lambda i: (0, 0)),
            pl.BlockSpec((1, D), lambda i: (0, 0)),
        ],
        out_specs=pl.BlockSpec((block_rows, D), lambda i: (i, 0)),
        out_shape=jax.ShapeDtypeStruct((N, D), jnp.float32),
    )


def kernel(x, edge_index, W, b):
    N, D = x.shape
    E = edge_index.shape[1]
    NC, NS = _sc_dims()
    unit = NS * 640
    n_pad = ((N + unit - 1) // unit) * unit

    ei = edge_index.astype(jnp.int32)
    NW = NC * NS
    e_per_w = E // NW
    nx = N + 8

    e_per_s = E // NS
    eps_pad = ((e_per_s + 16 * _CHUNK - 1) // (16 * _CHUNK)) * (16 * _CHUNK)
    dpad = ((0, 0), (0, eps_pad - e_per_s))
    dstp = jnp.pad(
        ei[1].reshape(NS, e_per_s), dpad, constant_values=n_pad - 1
    ).reshape(NS, eps_pad // _CHUNK, _CHUNK)

    epw_pad = ((e_per_w + 8 * _CHUNK - 1) // (8 * _CHUNK)) * (8 * _CHUNK)
    mpad = ((0, 0), (0, epw_pad - e_per_w))
    src3 = jnp.pad(
        ei[0].reshape(NW, e_per_w), mpad, constant_values=N
    ).reshape(NW, epw_pad // _CHUNK, _CHUNK)
    dst3 = jnp.pad(
        ei[1].reshape(NW, e_per_w), mpad, constant_values=n_pad - 1
    ).reshape(NW, epw_pad // _CHUNK, _CHUNK)

    acc_part, xs2, dinv = _make_fused_kernel(E, N, n_pad, D)(x, dstp, src3, dst3)

    block_rows = 1000 if N % 1000 == 0 else 8
    out = _make_final_kernel(N, NC * nx, D, block_rows)(
        acc_part, xs2, dinv.reshape(n_pad, 1), W, b.reshape(1, D)
    )
    return out


# speedup vs baseline: 1.1200x; 1.1200x over previous
import functools

import jax
import jax.numpy as jnp
from jax import lax
from jax.experimental import pallas as pl
from jax.experimental.pallas import tpu as pltpu
from jax.experimental.pallas import tpu_sc as plsc

_CHUNK = 80


def _sc_dims():
    try:
        info = plsc.get_sparse_core_info()
        return info.num_cores, info.num_subcores
    except Exception:
        return 2, 16


def _fill_1d(ref, n, value):
    v = jnp.full((16,), value, jnp.float32)

    def body(j, c):
        ref[pl.ds(j * 16, 16)] = v
        return c

    lax.fori_loop(0, n // 16, body, 0)


def _zero_2d(ref, rows, cols):
    z = jnp.zeros((16,), jnp.float32)
    per_row = cols // 16

    def body(j, c):
        ref[j // per_row, pl.ds((j % per_row) * 16, 16)] = z
        return c

    lax.fori_loop(0, rows * per_row, body, 0)


def _make_prep_kernel(E, N, n_pad, D):
    NC, NS = _sc_dims()
    n_chunks = E // (NS * _CHUNK)
    stripe = n_pad // NS
    xrows = 320
    mesh = plsc.VectorSubcoreMesh(core_axis_name="c", subcore_axis_name="s")

    @functools.partial(
        pl.kernel,
        out_type=[
            jax.ShapeDtypeStruct((N, D), jnp.float32),
            jax.ShapeDtypeStruct((n_pad,), jnp.float32),
        ],
        mesh=mesh,
        scratch_types=[
            pltpu.VMEM((n_chunks, _CHUNK), jnp.int32),
            pltpu.VMEM((_CHUNK,), jnp.float32),
            pltpu.VMEM((stripe + 16,), jnp.float32),
            pltpu.VMEM((xrows, D), jnp.float32),
            pltpu.VMEM_SHARED((n_pad,), jnp.float32),
            pltpu.SemaphoreType.DMA,
        ],
    )
    def prep_kernel(x_hbm, dst2_hbm, xs_out, dinv_out,
                    idx_v, ones_v, dbuf_v, xbuf_v, deg_sh, sem):
        cid = lax.axis_index("c")
        sid = lax.axis_index("s")
        _fill_1d(dbuf_v, stripe + 16, 0.0)
        _fill_1d(ones_v, _CHUNK, 1.0)
        pltpu.sync_copy(dst2_hbm.at[sid], idx_v)
        pltpu.sync_copy(dbuf_v.at[pl.ds(0, stripe)], deg_sh.at[pl.ds(sid * stripe, stripe)])
        plsc.subcore_barrier()

        def body(i, c):
            pltpu.async_copy(ones_v, deg_sh.at[idx_v.at[i]], sem, add=True)
            return c

        lax.fori_loop(0, n_chunks, body, 0)
        rowstart = jnp.minimum(sid * stripe + cid * xrows, N - xrows)
        pltpu.sync_copy(x_hbm.at[pl.ds(rowstart, xrows)], xbuf_v)

        def drain(i, c):
            pltpu.make_async_copy(ones_v, deg_sh.at[idx_v.at[0]], sem).wait()
            return c

        lax.fori_loop(0, n_chunks, drain, 0)
        plsc.subcore_barrier()

        pltpu.sync_copy(deg_sh.at[pl.ds(sid * stripe, stripe)], dbuf_v.at[pl.ds(0, stripe)])

        def newton(j, c):
            d = dbuf_v[pl.ds(j * 16, 16)] + 1.0
            bits = lax.bitcast_convert_type(d, jnp.int32)
            y = lax.bitcast_convert_type(
                jnp.int32(0x5F3759DF) - (bits >> 1), jnp.float32
            )
            y = y * (1.5 - 0.5 * d * y * y)
            y = y * (1.5 - 0.5 * d * y * y)
            y = y * (1.5 - 0.5 * d * y * y)
            dbuf_v[pl.ds(j * 16, 16)] = y
            return c

        lax.fori_loop(0, stripe // 16, newton, 0)

        @pl.when(cid == 0)
        def _():
            pltpu.sync_copy(
                dbuf_v.at[pl.ds(0, stripe)],
                dinv_out.at[pl.ds(sid * stripe, stripe)],
            )

        dbase = rowstart - sid * stripe

        def srow(r, c):
            y = dbuf_v[pl.ds(dbase + r, 16)][0]
            for j in range(D // 16):
                xbuf_v[r, pl.ds(j * 16, 16)] = xbuf_v[r, pl.ds(j * 16, 16)] * y
            return c

        lax.fori_loop(0, xrows, srow, 0)
        pltpu.sync_copy(xbuf_v, xs_out.at[pl.ds(rowstart, xrows)])

    return prep_kernel


def _make_msg_kernel(E, n_pad, D):
    NC, NS = _sc_dims()
    NW = NC * NS
    epw_pad = ((E // NW + 127) // 128) * 128
    n_chunks = epw_pad // 128
    half = 40
    stripe = n_pad // NS
    mesh = plsc.VectorSubcoreMesh(core_axis_name="c", subcore_axis_name="s")

    @functools.partial(
        pl.kernel,
        out_type=jax.ShapeDtypeStruct((NC, n_pad, D), jnp.float32),
        mesh=mesh,
        scratch_types=[
            pltpu.VMEM((min(half, n_chunks), 128), jnp.int32),
            pltpu.VMEM((n_chunks, 128), jnp.int32),
            pltpu.VMEM((2, 128, D), jnp.float32),
            pltpu.VMEM_SHARED((n_pad, D), jnp.float32),
            pltpu.SemaphoreType.DMA,
            pltpu.SemaphoreType.DMA,
            pltpu.SemaphoreType.DMA,
            pltpu.SemaphoreType.DMA,
        ],
    )
    def msg_kernel(xs_hbm, src3_hbm, dst3_hbm, out_hbm,
                   sidx_v, didx_v, rows_v, acc_sh, gsem0, gsem1, ssem0, ssem1):
        cid = lax.axis_index("c")
        sid = lax.axis_index("s")
        wid = cid * NS + sid
        row0 = sid * stripe
        pltpu.sync_copy(src3_hbm.at[wid, pl.ds(0, min(half, n_chunks))], sidx_v)
        pltpu.sync_copy(dst3_hbm.at[wid], didx_v)
        pltpu.async_copy(
            xs_hbm.at[sidx_v.at[0, pl.ds(0, 64)]],
            rows_v.at[1, pl.ds(0, 64)],
            gsem1,
        )
        pltpu.async_copy(
            xs_hbm.at[sidx_v.at[0, pl.ds(64, 64)]],
            rows_v.at[1, pl.ds(64, 64)],
            gsem1,
        )
        zrows = 40
        _zero_2d(rows_v.at[0], zrows, D)

        def zbody(r, c):
            pltpu.sync_copy(
                rows_v.at[0, pl.ds(0, zrows)],
                acc_sh.at[pl.ds(row0 + r * zrows, zrows)],
            )
            return c

        lax.fori_loop(0, stripe // zrows, zbody, 0)
        plsc.subcore_barrier()

        def gfire(i, buf, gsem):
            r = i % half
            pltpu.async_copy(
                xs_hbm.at[sidx_v.at[r, pl.ds(0, 64)]],
                rows_v.at[buf, pl.ds(0, 64)],
                gsem,
            )
            pltpu.async_copy(
                xs_hbm.at[sidx_v.at[r, pl.ds(64, 64)]],
                rows_v.at[buf, pl.ds(64, 64)],
                gsem,
            )

        def gwait(buf, gsem):
            for _q in range(2):
                pltpu.make_async_copy(
                    xs_hbm.at[sidx_v.at[0, pl.ds(0, 64)]],
                    rows_v.at[buf, pl.ds(0, 64)],
                    gsem,
                ).wait()

        def sfire(i, buf, ssem):
            pltpu.async_copy(
                rows_v.at[buf], acc_sh.at[didx_v.at[i]], ssem, add=True
            )

        def sdrain(buf, ssem):
            pltpu.make_async_copy(
                rows_v.at[buf], acc_sh.at[didx_v.at[0]], ssem
            ).wait()

        def halfstep(i, cur, cgsem, cssem, nxt, ngsem, nssem):
            @pl.when(i >= 1)
            def _():
                sdrain(nxt, nssem)

            @pl.when((i + 1 < n_chunks) & (i + 1 != half))
            def _():
                gfire(i + 1, nxt, ngsem)

            gwait(cur, cgsem)
            sfire(i, cur, cssem)

            @pl.when((i + 1 == half) & (i + 1 < n_chunks))
            def _():
                pltpu.sync_copy(
                    src3_hbm.at[wid, pl.ds(half, n_chunks - half)],
                    sidx_v.at[pl.ds(0, n_chunks - half)],
                )
                gfire(i + 1, nxt, ngsem)

        def body(k, c):
            halfstep(2 * k, 1, gsem1, ssem1, 0, gsem0, ssem0)

            @pl.when(2 * k + 1 < n_chunks)
            def _():
                halfstep(2 * k + 1, 0, gsem0, ssem0, 1, gsem1, ssem1)

            return c

        lax.fori_loop(0, (n_chunks + 1) // 2, body, 0)
        sdrain((n_chunks - 1) % 2, ssem0 if (n_chunks - 1) % 2 == 1 else ssem1)
        plsc.subcore_barrier()
        pltpu.sync_copy(
            acc_sh.at[pl.ds(row0, stripe)],
            out_hbm.at[cid, pl.ds(row0, stripe)],
        )

    return msg_kernel


def _final_body(acc_ref, xs_ref, dinv_ref, w_ref, b_ref, out_ref):
    h = (acc_ref[0] + acc_ref[1] + xs_ref[...]) * dinv_ref[...]
    out_ref[...] = (
        jnp.dot(h, w_ref[...], preferred_element_type=jnp.float32) + b_ref[...]
    )


def _make_final_kernel(N, D, block_rows):
    return pl.pallas_call(
        _final_body,
        grid=(N // block_rows,),
        in_specs=[
            pl.BlockSpec((2, block_rows, D), lambda i: (0, i, 0)),
            pl.BlockSpec((block_rows, D), lambda i: (i, 0)),
            pl.BlockSpec((block_rows, 1), lambda i: (i, 0)),
            pl.BlockSpec((D, D), lambda i: (0, 0)),
            pl.BlockSpec((1, D), lambda i: (0, 0)),
        ],
        out_specs=pl.BlockSpec((block_rows, D), lambda i: (i, 0)),
        out_shape=jax.ShapeDtypeStruct((N, D), jnp.float32),
    )


def kernel(x, edge_index, W, b):
    N, D = x.shape
    E = edge_index.shape[1]
    NC, NS = _sc_dims()
    unit = NS * _CHUNK
    n_pad = ((N + unit - 1) // unit) * unit

    ei = edge_index.astype(jnp.int32)
    NW = NC * NS
    e_per_w = E // NW
    dst2d = ei[1].reshape(NS, E // NS // _CHUNK, _CHUNK)
    epw_pad = ((e_per_w + 127) // 128) * 128
    pad = ((0, 0), (0, epw_pad - e_per_w))
    src3 = jnp.pad(ei[0].reshape(NW, e_per_w), pad).reshape(
        NW, epw_pad // 128, 128
    )
    dst3 = jnp.pad(
        ei[1].reshape(NW, e_per_w), pad, constant_values=n_pad - 1
    ).reshape(NW, epw_pad // 128, 128)
    xs, dinv = _make_prep_kernel(E, N, n_pad, D)(x, dst2d)

    acc_part = _make_msg_kernel(E, n_pad, D)(xs, src3, dst3)

    block_rows = 1000 if N % 1000 == 0 else 8
    out = _make_final_kernel(N, D, block_rows)(
        acc_part, xs, dinv.reshape(n_pad, 1), W, b.reshape(1, D)
    )
    return out
